# baseline (speedup 1.0000x reference)
import functools
import math

import jax
import jax.numpy as jnp
from jax import lax
from jax.experimental import pallas as pl
from jax.experimental.pallas import tpu as pltpu
from jax.experimental.pallas import tpu_sc as plsc

N_NODES = 10000
N_EDGES = 160000
NUM_GRAPHS = 16

NC = 2
NS = 16
NW = NC * NS
CHUNK = 128
EPAD = 163840
TPT = EPAD // NW
NCHUNK = TPT // CHUNK
DH = 128
RPT = N_NODES // NS
RB = 400


def _mesh():
    return plsc.VectorSubcoreMesh(
        core_axis_name="c", subcore_axis_name="s", num_cores=NC,
        num_subcores=NS)


@functools.lru_cache(maxsize=None)
def _mm_fn(k_dim, m_dim):
    grid = N_NODES // RB

    def body(x_ref, w_ref, b_ref, o_ref):
        o_ref[...] = jnp.dot(
            x_ref[...], w_ref[...],
            preferred_element_type=jnp.float32) + b_ref[0:1, :]

    return pl.pallas_call(
        body,
        grid=(grid,),
        in_specs=[
            pl.BlockSpec((RB, k_dim), lambda i: (i, 0)),
            pl.BlockSpec((k_dim, m_dim), lambda i: (0, 0)),
            pl.BlockSpec((8, m_dim), lambda i: (0, 0)),
        ],
        out_specs=pl.BlockSpec((RB, m_dim), lambda i: (i, 0)),
        out_shape=jax.ShapeDtypeStruct((N_NODES, m_dim), jnp.float32),
    )


def _mm(x, w, b):
    b8 = jnp.broadcast_to(b[None, :], (8, b.shape[0]))
    return _mm_fn(x.shape[1], w.shape[1])(x, w, b8)


@functools.lru_cache(maxsize=None)
def _alpha_fn(d):
    inv = 1.0 / math.sqrt(float(d))

    @functools.partial(
        pl.kernel,
        out_type=jax.ShapeDtypeStruct((EPAD,), jnp.float32),
        mesh=_mesh(),
        scratch_types=[
            pltpu.VMEM((CHUNK,), jnp.int32),
            pltpu.VMEM((CHUNK,), jnp.int32),
            pltpu.VMEM((CHUNK, d), jnp.float32),
            pltpu.VMEM((CHUNK, d), jnp.float32),
            pltpu.VMEM((CHUNK,), jnp.float32),
            pltpu.SemaphoreType.DMA,
        ],
        compiler_params=pltpu.CompilerParams(use_tc_tiling_on_sc=False, needs_layout_passes=False),
    )
    def kern(q_hbm, k_hbm, src_hbm, dst_hbm, e_hbm,
             didx, sidx, qbuf, kbuf, ebuf, sem):
        cid = lax.axis_index("c")
        sid = lax.axis_index("s")
        base = (sid * NC + cid) * TPT
        iota = lax.iota(jnp.int32, 16)

        def chunk(ci, carry):
            off = base + ci * CHUNK
            pltpu.sync_copy(dst_hbm.at[pl.ds(off, CHUNK)], didx)
            pltpu.sync_copy(src_hbm.at[pl.ds(off, CHUNK)], sidx)
            dq = pltpu.async_copy(q_hbm.at[didx], qbuf, sem)
            dk = pltpu.async_copy(k_hbm.at[sidx], kbuf, sem)
            dq.wait()
            dk.wait()

            def grp(g, carry2):
                eids = g * 16 + iota

                def dot(j, acc):
                    jv = jnp.full((16,), j, jnp.int32)
                    qv = plsc.load_gather(qbuf, [eids, jv])
                    kv = plsc.load_gather(kbuf, [eids, jv])
                    return acc + qv * kv

                acc = lax.fori_loop(0, d, dot, jnp.zeros((16,), jnp.float32),
                                    unroll=16)
                e = jnp.exp(acc * inv)
                e = jnp.where(off + eids < N_EDGES, e, 0.0)
                ebuf[pl.ds(g * 16, 16)] = e
                return carry2

            lax.fori_loop(0, CHUNK // 16, grp, 0)
            pltpu.sync_copy(ebuf, e_hbm.at[pl.ds(off, CHUNK)])
            return carry

        lax.fori_loop(0, NCHUNK, chunk, 0)

    return kern


@functools.lru_cache(maxsize=None)
def _agg_fn():
    @functools.partial(
        pl.kernel,
        out_type=(
            jax.ShapeDtypeStruct((NC, N_NODES, DH), jnp.float32),
            jax.ShapeDtypeStruct((NC, N_NODES, 16), jnp.float32),
        ),
        mesh=_mesh(),
        scratch_types=[
            pltpu.VMEM((CHUNK,), jnp.int32),
            pltpu.VMEM((CHUNK,), jnp.int32),
            pltpu.VMEM((CHUNK, DH), jnp.float32),
            pltpu.VMEM((CHUNK,), jnp.float32),
            pltpu.VMEM((CHUNK, 16), jnp.float32),
            pltpu.VMEM_SHARED((N_NODES, DH), jnp.float32),
            pltpu.VMEM_SHARED((N_NODES, 16), jnp.float32),
            pltpu.SemaphoreType.DMA,
        ],
        compiler_params=pltpu.CompilerParams(use_tc_tiling_on_sc=False, needs_layout_passes=False),
    )
    def kern(v_hbm, src_hbm, dst_hbm, e_hbm, zrow_hbm, zden_hbm,
             agg_out, den_out,
             didx, sidx, vbuf, ebuf, dbuf, agg_s, den_s, sem):
        cid = lax.axis_index("c")
        sid = lax.axis_index("s")
        base = (sid * NC + cid) * TPT
        r0 = sid * RPT
        lane0 = jnp.where(lax.iota(jnp.int32, 16) == 0, 1.0, 0.0)

        pltpu.sync_copy(zrow_hbm, agg_s.at[pl.ds(r0, RPT)])
        pltpu.sync_copy(zden_hbm, den_s.at[pl.ds(r0, RPT)])
        plsc.subcore_barrier()

        def chunk(ci, carry):
            off = base + ci * CHUNK
            pltpu.sync_copy(dst_hbm.at[pl.ds(off, CHUNK)], didx)
            pltpu.sync_copy(src_hbm.at[pl.ds(off, CHUNK)], sidx)
            pltpu.sync_copy(e_hbm.at[pl.ds(off, CHUNK)], ebuf)
            pltpu.async_copy(v_hbm.at[sidx], vbuf, sem).wait()

            def row(i, carry2):
                iv = jnp.full((16,), i, jnp.int32)
                es = plsc.load_gather(ebuf, [iv])
                for j in range(DH // 16):
                    vbuf[i, pl.ds(j * 16, 16)] = (
                        vbuf[i, pl.ds(j * 16, 16)] * es)
                dbuf[i, :] = es * lane0
                return carry2

            lax.fori_loop(0, CHUNK, row, 0, unroll=4)
            pltpu.sync_copy(vbuf, agg_s.at[didx], add=True)
            pltpu.sync_copy(dbuf, den_s.at[didx], add=True)
            return carry

        lax.fori_loop(0, NCHUNK, chunk, 0)
        plsc.subcore_barrier()
        pltpu.sync_copy(agg_s.at[pl.ds(r0, RPT)],
                        agg_out.at[cid, pl.ds(r0, RPT)])
        pltpu.sync_copy(den_s.at[pl.ds(r0, RPT)],
                        den_out.at[cid, pl.ds(r0, RPT)])

    return kern


@functools.lru_cache(maxsize=None)
def _combine_fn(d, elu):
    grid = N_NODES // RB
    nh = d // DH

    def body(*refs):
        agg_refs = refs[:nh]
        den_ref = refs[nh]
        s_ref = refs[nh + 1]
        o_ref = refs[nh + 2]
        den = den_ref[0, :, 0:1] + den_ref[1, :, 0:1] + 1e-16
        parts = [a[0] + a[1] for a in (r[...] for r in agg_refs)]
        u = parts[0] if nh == 1 else jnp.concatenate(parts, axis=1)
        out = u / den + s_ref[...]
        if elu:
            out = jnp.where(out > 0, out, jnp.exp(jnp.minimum(out, 0.0)) - 1.0)
        o_ref[...] = out

    in_specs = (
        [pl.BlockSpec((NC, RB, DH), lambda i: (0, i, 0)) for _ in range(nh)]
        + [pl.BlockSpec((NC, RB, 16), lambda i: (0, i, 0)),
           pl.BlockSpec((RB, d), lambda i: (i, 0))]
    )
    return pl.pallas_call(
        body,
        grid=(grid,),
        in_specs=in_specs,
        out_specs=pl.BlockSpec((RB, d), lambda i: (i, 0)),
        out_shape=jax.ShapeDtypeStruct((N_NODES, d), jnp.float32),
    )


@functools.lru_cache(maxsize=None)
def _pool_fn(d):
    grid = N_NODES // RB

    def body(x_ref, b_ref, o_ref, cnt_ref):
        k = pl.program_id(0)

        @pl.when(k == 0)
        def _():
            o_ref[...] = jnp.zeros_like(o_ref)
            cnt_ref[...] = jnp.zeros_like(cnt_ref)

        x = x_ref[...]
        b = b_ref[...]
        rows = []
        cnts = []
        for g in range(NUM_GRAPHS):
            m = b == g
            rows.append(jnp.sum(jnp.where(m, x, 0.0), axis=0))
            cnts.append(jnp.sum(m.astype(jnp.float32), axis=0))
        o_ref[...] += jnp.stack(rows)
        cnt_ref[...] += jnp.stack(cnts)

        @pl.when(k == grid - 1)
        def _():
            o_ref[...] = o_ref[...] / jnp.maximum(cnt_ref[...], 1.0)

    return pl.pallas_call(
        body,
        grid=(grid,),
        in_specs=[
            pl.BlockSpec((RB, d), lambda i: (i, 0)),
            pl.BlockSpec((RB, d), lambda i: (i, 0)),
        ],
        out_specs=pl.BlockSpec((NUM_GRAPHS, d), lambda i: (0, 0)),
        out_shape=jax.ShapeDtypeStruct((NUM_GRAPHS, d), jnp.float32),
        scratch_shapes=[pltpu.VMEM((NUM_GRAPHS, d), jnp.float32)],
    )


def _tconv(p, x, srcp, dstp, zrow, zden, elu):
    d = p["Wq"].shape[1]
    w_cat = jnp.concatenate([p["Wq"], p["Wk"], p["Wv"], p["Ws"]], axis=1)
    b_cat = jnp.concatenate([p["bq"], p["bk"], p["bv"], p["bs"]])
    qkvs = _mm(x, w_cat, b_cat)
    q = qkvs[:, :d]
    k = qkvs[:, d:2 * d]
    v = qkvs[:, 2 * d:3 * d]
    s = qkvs[:, 3 * d:]
    e = _alpha_fn(d)(q, k, srcp, dstp)
    halves = []
    den = None
    for h in range(d // DH):
        vh = v[:, h * DH:(h + 1) * DH]
        agg, dn = _agg_fn()(vh, srcp, dstp, e, zrow, zden)
        halves.append(agg)
        if h == 0:
            den = dn
    return _combine_fn(d, elu)(*halves, den, s)


def kernel(features, img_feat, edge_index, batch_index, params):
    src = edge_index[0]
    dst = edge_index[1]
    pad = EPAD - N_EDGES
    srcp = jnp.concatenate([src, jnp.zeros((pad,), jnp.int32)])
    dstp = jnp.concatenate([dst, jnp.zeros((pad,), jnp.int32)])
    zrow = jnp.zeros((RPT, DH), jnp.float32)
    zden = jnp.zeros((RPT, 16), jnp.float32)

    def conv(name, x, elu):
        return _tconv(params[name], x, srcp, dstp, zrow, zden, elu)

    h1 = conv("conv1", features, True)
    h2 = conv("conv2", h1, False)
    h3 = conv("conv3", h2, True)
    h4 = conv("conv4", h3, False)
    img1 = conv("imgconv1", img_feat, True)
    img2 = conv("imgconv2", img1, False)
    img3 = conv("imgconv3", img2, True)
    img4 = conv("imgconv4", img3, False)
    concat = jnp.concatenate([h2, img2], axis=1)
    combine = conv("neck", concat, True)
    c2 = conv("neck2", combine, False)
    c3 = conv("c3", c2, True)
    c4 = conv("c4", c3, False)

    bb = jnp.broadcast_to(batch_index[:, None], (N_NODES, c2.shape[1]))
    hidden = _pool_fn(c2.shape[1])(c2, bb.astype(jnp.int32))
    return (h2, img2, c2, h4, img4, c4, hidden)


# speedup vs baseline: 1.0767x; 1.0767x over previous
import functools
import math

import jax
import jax.numpy as jnp
from jax import lax
from jax.experimental import pallas as pl
from jax.experimental.pallas import tpu as pltpu
from jax.experimental.pallas import tpu_sc as plsc

N_NODES = 10000
N_EDGES = 160000
NUM_GRAPHS = 16

NC = 2
NS = 16
NW = NC * NS
CHUNK = 128
EPAD = 163840
TPT = EPAD // NW
NCHUNK = TPT // CHUNK
DH = 128
RPT = N_NODES // NS
RB = 400


def _mesh():
    return plsc.VectorSubcoreMesh(
        core_axis_name="c", subcore_axis_name="s", num_cores=NC,
        num_subcores=NS)


@functools.lru_cache(maxsize=None)
def _mm_fn(k_dim, m_dim):
    grid = N_NODES // RB

    def body(x_ref, w_ref, b_ref, o_ref):
        o_ref[...] = jnp.dot(
            x_ref[...], w_ref[...],
            preferred_element_type=jnp.float32) + b_ref[0:1, :]

    return pl.pallas_call(
        body,
        grid=(grid,),
        in_specs=[
            pl.BlockSpec((RB, k_dim), lambda i: (i, 0)),
            pl.BlockSpec((k_dim, m_dim), lambda i: (0, 0)),
            pl.BlockSpec((8, m_dim), lambda i: (0, 0)),
        ],
        out_specs=pl.BlockSpec((RB, m_dim), lambda i: (i, 0)),
        out_shape=jax.ShapeDtypeStruct((N_NODES, m_dim), jnp.float32),
    )


def _mm(x, w, b):
    b8 = jnp.broadcast_to(b[None, :], (8, b.shape[0]))
    return _mm_fn(x.shape[1], w.shape[1])(x, w, b8)


CA = 64
NPAIR_A = TPT // (2 * CA)


@functools.lru_cache(maxsize=None)
def _alpha_fn(d):
    inv = 1.0 / math.sqrt(float(d))

    @functools.partial(
        pl.kernel,
        out_type=jax.ShapeDtypeStruct((EPAD,), jnp.float32),
        mesh=_mesh(),
        scratch_types=[
            pltpu.VMEM((CA,), jnp.int32),
            pltpu.VMEM((CA,), jnp.int32),
            pltpu.VMEM((CA,), jnp.int32),
            pltpu.VMEM((CA,), jnp.int32),
            pltpu.VMEM((CA, d), jnp.float32),
            pltpu.VMEM((CA, d), jnp.float32),
            pltpu.VMEM((CA, d), jnp.float32),
            pltpu.VMEM((CA, d), jnp.float32),
            pltpu.VMEM((2 * CA,), jnp.float32),
            pltpu.SemaphoreType.DMA,
        ],
        compiler_params=pltpu.CompilerParams(use_tc_tiling_on_sc=False, needs_layout_passes=False),
    )
    def kern(q_hbm, k_hbm, src_hbm, dst_hbm, e_hbm,
             didx0, sidx0, didx1, sidx1, qb0, kb0, qb1, kb1, ebuf, sem):
        cid = lax.axis_index("c")
        sid = lax.axis_index("s")
        base = (sid * NC + cid) * TPT
        iota = lax.iota(jnp.int32, 16)

        def compute(qb, kb, off, elo):
            def grp(g, carry2):
                eids = g * 16 + iota

                def dot(j, acc):
                    jv = jnp.full((16,), j, jnp.int32)
                    qv = plsc.load_gather(qb, [eids, jv])
                    kv = plsc.load_gather(kb, [eids, jv])
                    return acc + qv * kv

                acc = lax.fori_loop(0, d, dot, jnp.zeros((16,), jnp.float32),
                                    unroll=16)
                e = jnp.exp(acc * inv)
                e = jnp.where(off + eids < N_EDGES, e, 0.0)
                ebuf[pl.ds(elo + g * 16, 16)] = e
                return carry2

            lax.fori_loop(0, CA // 16, grp, 0)

        def pair(p, carry):
            off0 = base + p * 2 * CA
            off1 = off0 + CA
            w = [
                pltpu.async_copy(dst_hbm.at[pl.ds(off0, CA)], didx0, sem),
                pltpu.async_copy(src_hbm.at[pl.ds(off0, CA)], sidx0, sem),
                pltpu.async_copy(dst_hbm.at[pl.ds(off1, CA)], didx1, sem),
                pltpu.async_copy(src_hbm.at[pl.ds(off1, CA)], sidx1, sem),
            ]
            for c in w:
                c.wait()
            g = [
                pltpu.async_copy(q_hbm.at[didx0], qb0, sem),
                pltpu.async_copy(k_hbm.at[sidx0], kb0, sem),
                pltpu.async_copy(q_hbm.at[didx1], qb1, sem),
                pltpu.async_copy(k_hbm.at[sidx1], kb1, sem),
            ]
            for c in g:
                c.wait()
            compute(qb0, kb0, off0, 0)
            compute(qb1, kb1, off1, CA)
            pltpu.sync_copy(ebuf, e_hbm.at[pl.ds(off0, 2 * CA)])
            return carry

        lax.fori_loop(0, NPAIR_A, pair, 0)

    return kern


@functools.lru_cache(maxsize=None)
def _agg_fn():
    @functools.partial(
        pl.kernel,
        out_type=(
            jax.ShapeDtypeStruct((NC, N_NODES, DH), jnp.float32),
            jax.ShapeDtypeStruct((NC, N_NODES, 16), jnp.float32),
        ),
        mesh=_mesh(),
        scratch_types=[
            pltpu.VMEM((CHUNK,), jnp.int32),
            pltpu.VMEM((CHUNK,), jnp.int32),
            pltpu.VMEM((CHUNK,), jnp.int32),
            pltpu.VMEM((CHUNK,), jnp.int32),
            pltpu.VMEM((CHUNK, DH), jnp.float32),
            pltpu.VMEM((CHUNK, DH), jnp.float32),
            pltpu.VMEM((CHUNK,), jnp.float32),
            pltpu.VMEM((CHUNK,), jnp.float32),
            pltpu.VMEM((CHUNK, 16), jnp.float32),
            pltpu.VMEM((CHUNK, 16), jnp.float32),
            pltpu.VMEM_SHARED((N_NODES, DH), jnp.float32),
            pltpu.VMEM_SHARED((N_NODES, 16), jnp.float32),
            pltpu.SemaphoreType.DMA,
        ],
        compiler_params=pltpu.CompilerParams(use_tc_tiling_on_sc=False, needs_layout_passes=False),
    )
    def kern(v_hbm, src_hbm, dst_hbm, e_hbm, zrow_hbm, zden_hbm,
             agg_out, den_out,
             didx0, sidx0, didx1, sidx1, vb0, vb1, eb0, eb1, db0, db1,
             agg_s, den_s, sem):
        cid = lax.axis_index("c")
        sid = lax.axis_index("s")
        base = (sid * NC + cid) * TPT
        r0 = sid * RPT
        lane0 = jnp.where(lax.iota(jnp.int32, 16) == 0, 1.0, 0.0)

        pltpu.sync_copy(zrow_hbm, agg_s.at[pl.ds(r0, RPT)])
        pltpu.sync_copy(zden_hbm, den_s.at[pl.ds(r0, RPT)])
        plsc.subcore_barrier()

        def scale(vb, eb, db):
            def row(i, carry2):
                iv = jnp.full((16,), i, jnp.int32)
                es = plsc.load_gather(eb, [iv])
                for j in range(DH // 16):
                    vb[i, pl.ds(j * 16, 16)] = (
                        vb[i, pl.ds(j * 16, 16)] * es)
                db[i, :] = es * lane0
                return carry2

            lax.fori_loop(0, CHUNK, row, 0, unroll=4)

        def pair(p, carry):
            off0 = base + p * 2 * CHUNK
            off1 = off0 + CHUNK
            w = [
                pltpu.async_copy(dst_hbm.at[pl.ds(off0, CHUNK)], didx0, sem),
                pltpu.async_copy(src_hbm.at[pl.ds(off0, CHUNK)], sidx0, sem),
                pltpu.async_copy(e_hbm.at[pl.ds(off0, CHUNK)], eb0, sem),
                pltpu.async_copy(dst_hbm.at[pl.ds(off1, CHUNK)], didx1, sem),
                pltpu.async_copy(src_hbm.at[pl.ds(off1, CHUNK)], sidx1, sem),
                pltpu.async_copy(e_hbm.at[pl.ds(off1, CHUNK)], eb1, sem),
            ]
            for c in w:
                c.wait()
            g = [
                pltpu.async_copy(v_hbm.at[sidx0], vb0, sem),
                pltpu.async_copy(v_hbm.at[sidx1], vb1, sem),
            ]
            for c in g:
                c.wait()
            scale(vb0, eb0, db0)
            scale(vb1, eb1, db1)
            s = [
                pltpu.async_copy(vb0, agg_s.at[didx0], sem, add=True),
                pltpu.async_copy(db0, den_s.at[didx0], sem, add=True),
                pltpu.async_copy(vb1, agg_s.at[didx1], sem, add=True),
                pltpu.async_copy(db1, den_s.at[didx1], sem, add=True),
            ]
            for c in s:
                c.wait()
            return carry

        lax.fori_loop(0, NCHUNK // 2, pair, 0)
        plsc.subcore_barrier()
        pltpu.sync_copy(agg_s.at[pl.ds(r0, RPT)],
                        agg_out.at[cid, pl.ds(r0, RPT)])
        pltpu.sync_copy(den_s.at[pl.ds(r0, RPT)],
                        den_out.at[cid, pl.ds(r0, RPT)])

    return kern


@functools.lru_cache(maxsize=None)
def _combine_fn(d, elu):
    grid = N_NODES // RB
    nh = d // DH

    def body(*refs):
        agg_refs = refs[:nh]
        den_ref = refs[nh]
        s_ref = refs[nh + 1]
        o_ref = refs[nh + 2]
        den = den_ref[0, :, 0:1] + den_ref[1, :, 0:1] + 1e-16
        parts = [a[0] + a[1] for a in (r[...] for r in agg_refs)]
        u = parts[0] if nh == 1 else jnp.concatenate(parts, axis=1)
        out = u / den + s_ref[...]
        if elu:
            out = jnp.where(out > 0, out, jnp.exp(jnp.minimum(out, 0.0)) - 1.0)
        o_ref[...] = out

    in_specs = (
        [pl.BlockSpec((NC, RB, DH), lambda i: (0, i, 0)) for _ in range(nh)]
        + [pl.BlockSpec((NC, RB, 16), lambda i: (0, i, 0)),
           pl.BlockSpec((RB, d), lambda i: (i, 0))]
    )
    return pl.pallas_call(
        body,
        grid=(grid,),
        in_specs=in_specs,
        out_specs=pl.BlockSpec((RB, d), lambda i: (i, 0)),
        out_shape=jax.ShapeDtypeStruct((N_NODES, d), jnp.float32),
    )


@functools.lru_cache(maxsize=None)
def _pool_fn(d):
    grid = N_NODES // RB

    def body(x_ref, b_ref, o_ref, cnt_ref):
        k = pl.program_id(0)

        @pl.when(k == 0)
        def _():
            o_ref[...] = jnp.zeros_like(o_ref)
            cnt_ref[...] = jnp.zeros_like(cnt_ref)

        x = x_ref[...]
        b = b_ref[...]
        rows = []
        cnts = []
        for g in range(NUM_GRAPHS):
            m = b == g
            rows.append(jnp.sum(jnp.where(m, x, 0.0), axis=0))
            cnts.append(jnp.sum(m.astype(jnp.float32), axis=0))
        o_ref[...] += jnp.stack(rows)
        cnt_ref[...] += jnp.stack(cnts)

        @pl.when(k == grid - 1)
        def _():
            o_ref[...] = o_ref[...] / jnp.maximum(cnt_ref[...], 1.0)

    return pl.pallas_call(
        body,
        grid=(grid,),
        in_specs=[
            pl.BlockSpec((RB, d), lambda i: (i, 0)),
            pl.BlockSpec((RB, d), lambda i: (i, 0)),
        ],
        out_specs=pl.BlockSpec((NUM_GRAPHS, d), lambda i: (0, 0)),
        out_shape=jax.ShapeDtypeStruct((NUM_GRAPHS, d), jnp.float32),
        scratch_shapes=[pltpu.VMEM((NUM_GRAPHS, d), jnp.float32)],
    )


def _tconv(p, x, srcp, dstp, zrow, zden, elu):
    d = p["Wq"].shape[1]
    w_cat = jnp.concatenate([p["Wq"], p["Wk"], p["Wv"], p["Ws"]], axis=1)
    b_cat = jnp.concatenate([p["bq"], p["bk"], p["bv"], p["bs"]])
    qkvs = _mm(x, w_cat, b_cat)
    q = qkvs[:, :d]
    k = qkvs[:, d:2 * d]
    v = qkvs[:, 2 * d:3 * d]
    s = qkvs[:, 3 * d:]
    e = _alpha_fn(d)(q, k, srcp, dstp)
    halves = []
    den = None
    for h in range(d // DH):
        vh = v[:, h * DH:(h + 1) * DH]
        agg, dn = _agg_fn()(vh, srcp, dstp, e, zrow, zden)
        halves.append(agg)
        if h == 0:
            den = dn
    return _combine_fn(d, elu)(*halves, den, s)


def kernel(features, img_feat, edge_index, batch_index, params):
    src = edge_index[0]
    dst = edge_index[1]
    pad = EPAD - N_EDGES
    srcp = jnp.concatenate([src, jnp.zeros((pad,), jnp.int32)])
    dstp = jnp.concatenate([dst, jnp.zeros((pad,), jnp.int32)])
    zrow = jnp.zeros((RPT, DH), jnp.float32)
    zden = jnp.zeros((RPT, 16), jnp.float32)

    def conv(name, x, elu):
        return _tconv(params[name], x, srcp, dstp, zrow, zden, elu)

    h1 = conv("conv1", features, True)
    h2 = conv("conv2", h1, False)
    h3 = conv("conv3", h2, True)
    h4 = conv("conv4", h3, False)
    img1 = conv("imgconv1", img_feat, True)
    img2 = conv("imgconv2", img1, False)
    img3 = conv("imgconv3", img2, True)
    img4 = conv("imgconv4", img3, False)
    concat = jnp.concatenate([h2, img2], axis=1)
    combine = conv("neck", concat, True)
    c2 = conv("neck2", combine, False)
    c3 = conv("c3", c2, True)
    c4 = conv("c4", c3, False)

    bb = jnp.broadcast_to(batch_index[:, None], (N_NODES, c2.shape[1]))
    hidden = _pool_fn(c2.shape[1])(c2, bb.astype(jnp.int32))
    return (h2, img2, c2, h4, img4, c4, hidden)
